# baseline (speedup 1.0000x reference)
import functools

import jax
import jax.numpy as jnp
from jax import lax
from jax.experimental import pallas as pl
from jax.experimental.pallas import tpu as pltpu
from jax.experimental.pallas import tpu_sc as plsc

_B = 4
_SEQ = 2048
_D = 1024
_NP = 2
_NI = 1024
_NC = 2
_NS = 16
_NW = _NC * _NS
_SEG = _NW // _B
_IBLK = _NI // _SEG
_SLAB = 8
_NCHUNK = _IBLK // _SLAB
_NBUF = 6
_LOOK = 2


def _body(inp_hbm, idx_hbm, out_hbm, idx_v, buf_v, gsem, ssem):
    wid = lax.axis_index("s") * _NC + lax.axis_index("c")
    b = wid // _SEG
    i0 = (wid % _SEG) * _IBLK

    pltpu.sync_copy(idx_hbm.at[pl.ds(i0, _IBLK)], idx_v)

    tab = inp_hbm.at[b]

    def out_block(c):
        return out_hbm.at[b, pl.ds(i0 + c * _SLAB, _SLAB)]

    def drain(ref, sem):
        pltpu.make_async_copy(out_block(0), ref, sem).wait()

    def start_gathers(c):
        slot = c % _NBUF

        def start_pair(j, carry):
            pltpu.async_copy(
                tab.at[idx_v.at[c * _SLAB + j]], buf_v.at[slot, j], gsem
            )
            return carry

        lax.fori_loop(0, _SLAB, start_pair, 0, unroll=4)

    for k in range(_LOOK):
        start_gathers(k)
    for c in range(_NCHUNK):
        if c + _LOOK < _NCHUNK:
            if c + _LOOK >= _NBUF:
                drain(buf_v.at[(c + _LOOK) % _NBUF], ssem)
            start_gathers(c + _LOOK)
        drain(buf_v.at[c % _NBUF], gsem)
        pltpu.async_copy(buf_v.at[c % _NBUF], out_block(c), ssem)

    for slot in range(_NBUF):
        drain(buf_v.at[slot], ssem)


@jax.jit
def _split_pathways(inputs, indices):
    call = functools.partial(
        pl.kernel,
        out_type=jax.ShapeDtypeStruct((_B, _NI, _NP, _D), jnp.float32),
        mesh=plsc.VectorSubcoreMesh(core_axis_name="c", subcore_axis_name="s"),
        scratch_types=[
            pltpu.VMEM((_IBLK, _NP), jnp.int32),
            pltpu.VMEM((_NBUF, _SLAB, _NP, _D), jnp.float32),
            pltpu.SemaphoreType.DMA,
            pltpu.SemaphoreType.DMA,
        ],
    )(_body)
    return call(inputs, indices)


def kernel(inputs, indices):
    return _split_pathways(inputs, indices)


# speedup vs baseline: 1.0114x; 1.0114x over previous
import functools

import jax
import jax.numpy as jnp
from jax import lax
from jax.experimental import pallas as pl
from jax.experimental.pallas import tpu as pltpu
from jax.experimental.pallas import tpu_sc as plsc

_B = 4
_SEQ = 2048
_D = 1024
_NP = 2
_NI = 1024
_NC = 2
_NS = 16
_NW = _NC * _NS
_SEG = _NW // _B
_IBLK = _NI // _SEG
_SLAB = 16
_NCHUNK = _IBLK // _SLAB
_NBUF = 3


def _body(inp_hbm, idx_hbm, out_hbm, idx_v, buf_v, gsem, ssem):
    wid = lax.axis_index("s") * _NC + lax.axis_index("c")
    b = wid // _SEG
    i0 = (wid % _SEG) * _IBLK

    pltpu.sync_copy(idx_hbm.at[pl.ds(i0, _IBLK)], idx_v)

    tab = inp_hbm.at[b]

    def out_block(c):
        return out_hbm.at[b, pl.ds(i0 + c * _SLAB, _SLAB)]

    def drain(ref, sem):
        pltpu.make_async_copy(out_block(0), ref, sem).wait()

    def start_gathers(c):
        slot = c % _NBUF

        def start_pair(j, carry):
            pltpu.async_copy(
                tab.at[idx_v.at[c * _SLAB + j]], buf_v.at[slot, j], gsem
            )
            return carry

        lax.fori_loop(0, _SLAB, start_pair, 0, unroll=8)

    start_gathers(0)
    start_gathers(1)
    for c in range(_NCHUNK):
        if c + 2 < _NCHUNK:
            if c + 2 >= _NBUF:
                drain(buf_v.at[(c + 2) % _NBUF], ssem)
            start_gathers(c + 2)
        drain(buf_v.at[c % _NBUF], gsem)
        pltpu.async_copy(buf_v.at[c % _NBUF], out_block(c), ssem)

    for slot in range(_NBUF):
        drain(buf_v.at[slot], ssem)


@jax.jit
def _split_pathways(inputs, indices):
    call = functools.partial(
        pl.kernel,
        out_type=jax.ShapeDtypeStruct((_B, _NI, _NP, _D), jnp.float32),
        mesh=plsc.VectorSubcoreMesh(core_axis_name="c", subcore_axis_name="s"),
        scratch_types=[
            pltpu.VMEM((_IBLK, _NP), jnp.int32),
            pltpu.VMEM((_NBUF, _SLAB, _NP, _D), jnp.float32),
            pltpu.SemaphoreType.DMA,
            pltpu.SemaphoreType.DMA,
        ],
    )(_body)
    return call(inputs, indices)


def kernel(inputs, indices):
    return _split_pathways(inputs, indices)
